# Initial kernel scaffold; baseline (speedup 1.0000x reference)
import jax
import jax.numpy as jnp
from jax.experimental import pallas as pl


def kernel(x, edge_index, W1, b1, W2, b2):
    raise NotImplementedError("write your pallas kernel here")


# speedup vs baseline: 13.4485x; 13.4485x over previous
import functools

import jax
import jax.numpy as jnp
from jax import lax
from jax.experimental import pallas as pl
from jax.experimental.pallas import tpu as pltpu
from jax.experimental.pallas import tpu_sc as plsc

N = 10000
E = 320000
D_IN = 128
D_HID = 128
D_OUT = 64

NC = 2
NS = 16
NW = NC * NS

CHUNK = 80
EPT = E // NW
ITERS = EPT // CHUNK
N_PAD = 10240
RPT = N_PAD // NS
ZROWS = 128
DEG_PAD = 10240
DPT = DEG_PAD // NS

_mesh = lambda: plsc.VectorSubcoreMesh(core_axis_name="c", subcore_axis_name="s")


@functools.partial(
    pl.kernel,
    out_type=[jax.ShapeDtypeStruct((DEG_PAD,), jnp.float32)] * 2,
    mesh=_mesh(),
    scratch_types=[
        pltpu.VMEM((CHUNK,), jnp.int32),
        pltpu.VMEM((CHUNK,), jnp.float32),
        pltpu.VMEM((DPT,), jnp.float32),
        pltpu.VMEM_SHARED((DEG_PAD,), jnp.float32),
    ],
)
def _sc_deg(dst_hbm, out0, out1, dst_v, ones_v, zero_v, acc):
    c = lax.axis_index("c")
    s = lax.axis_index("s")
    for i in range(CHUNK // 16):
        ones_v[pl.ds(i * 16, 16)] = jnp.ones((16,), jnp.float32)

    def zfill(i, carry):
        zero_v[pl.ds(i * 16, 16)] = jnp.zeros((16,), jnp.float32)
        return carry

    lax.fori_loop(0, DPT // 16, zfill, 0)
    pltpu.sync_copy(zero_v, acc.at[pl.ds(s * DPT, DPT)])
    plsc.subcore_barrier()

    base = (c * NS + s) * EPT

    def body(i, carry):
        pltpu.sync_copy(dst_hbm.at[pl.ds(base + i * CHUNK, CHUNK)], dst_v)
        pltpu.sync_copy(ones_v, acc.at[dst_v], add=True)
        return carry

    lax.fori_loop(0, ITERS, body, 0)
    plsc.subcore_barrier()

    @pl.when(c == 0)
    def _():
        pltpu.sync_copy(acc.at[pl.ds(s * DPT, DPT)], out0.at[pl.ds(s * DPT, DPT)])

    @pl.when(c == 1)
    def _():
        pltpu.sync_copy(acc.at[pl.ds(s * DPT, DPT)], out1.at[pl.ds(s * DPT, DPT)])


def _make_sc_agg(D):
    @functools.partial(
        pl.kernel,
        out_type=[jax.ShapeDtypeStruct((N_PAD, D), jnp.float32)] * 2,
        mesh=_mesh(),
        scratch_types=[
            pltpu.VMEM((CHUNK,), jnp.int32),
            pltpu.VMEM((CHUNK,), jnp.int32),
            pltpu.VMEM((CHUNK, D), jnp.float32),
            pltpu.VMEM((ZROWS, D), jnp.float32),
            pltpu.VMEM_SHARED((N_PAD, D), jnp.float32),
            pltpu.SemaphoreType.DMA,
        ],
    )
    def agg(h_hbm, src_hbm, dst_hbm, out0, out1, src_v, dst_v, rows_v, zrows_v,
            acc, sem):
        c = lax.axis_index("c")
        s = lax.axis_index("s")

        def zfill(r, carry):
            for j in range(D // 16):
                zrows_v[r, pl.ds(j * 16, 16)] = jnp.zeros((16,), jnp.float32)
            return carry

        lax.fori_loop(0, ZROWS, zfill, 0)
        for t in range(RPT // ZROWS):
            pltpu.sync_copy(zrows_v, acc.at[pl.ds(s * RPT + t * ZROWS, ZROWS)])
        plsc.subcore_barrier()

        base = (c * NS + s) * EPT

        def body(i, carry):
            b = base + i * CHUNK
            pltpu.sync_copy(src_hbm.at[pl.ds(b, CHUNK)], src_v)
            pltpu.sync_copy(dst_hbm.at[pl.ds(b, CHUNK)], dst_v)
            pltpu.async_copy(h_hbm.at[src_v], rows_v, sem).wait()
            pltpu.sync_copy(rows_v, acc.at[dst_v], add=True)
            return carry

        lax.fori_loop(0, ITERS, body, 0)
        plsc.subcore_barrier()

        @pl.when(c == 0)
        def _():
            pltpu.sync_copy(acc.at[pl.ds(s * RPT, RPT)], out0.at[pl.ds(s * RPT, RPT)])

        @pl.when(c == 1)
        def _():
            pltpu.sync_copy(acc.at[pl.ds(s * RPT, RPT)], out1.at[pl.ds(s * RPT, RPT)])

    return agg


_sc_agg128 = _make_sc_agg(D_HID)


def _dis(d0_ref, d1_ref):
    return lax.rsqrt(d0_ref[:N] + d1_ref[:N] + 1.0)


def _tc_b_body(d0_ref, d1_ref, x_ref, w1_ref, h1p_ref):
    dis = _dis(d0_ref, d1_ref)
    h = jnp.dot(x_ref[...], w1_ref[...], preferred_element_type=jnp.float32)
    h1p_ref[...] = h * dis[:, None]


def _tc_d_body(d0_ref, d1_ref, h1p_ref, p0_ref, p1_ref, b1_ref, g_ref):
    dis = _dis(d0_ref, d1_ref)
    agg = p0_ref[:N] + p1_ref[:N] + h1p_ref[...]
    h1 = jnp.maximum(agg * dis[:, None] + b1_ref[...], 0.0)
    g_ref[...] = h1 * dis[:, None]


def _tc_f_body(d0_ref, d1_ref, g_ref, q0_ref, q1_ref, w2_ref, b2_ref, out_ref):
    dis = _dis(d0_ref, d1_ref)
    agg = (q0_ref[:N] + q1_ref[:N] + g_ref[...]) * dis[:, None]
    out_ref[...] = (
        jnp.dot(agg, w2_ref[...], preferred_element_type=jnp.float32)
        + b2_ref[...]
    )


def kernel(x, edge_index, W1, b1, W2, b2):
    ei = edge_index.astype(jnp.int32)
    src, dst = ei[0], ei[1]

    d0, d1 = _sc_deg(dst)

    h1p = pl.pallas_call(
        _tc_b_body,
        out_shape=jax.ShapeDtypeStruct((N, D_HID), jnp.float32),
    )(d0, d1, x, W1)

    p0, p1 = _sc_agg128(h1p, src, dst)

    g = pl.pallas_call(
        _tc_d_body,
        out_shape=jax.ShapeDtypeStruct((N, D_HID), jnp.float32),
    )(d0, d1, h1p, p0, p1, b1)

    q0, q1 = _sc_agg128(g, src, dst)

    out = pl.pallas_call(
        _tc_f_body,
        out_shape=jax.ShapeDtypeStruct((N, D_OUT), jnp.float32),
    )(d0, d1, g, q0, q1, W2, b2)

    return out
